# baseline (speedup 1.0000x reference)
import functools

import jax
import jax.numpy as jnp
from jax import lax
from jax.experimental import pallas as pl
from jax.experimental.pallas import tpu as pltpu
from jax.experimental.pallas import tpu_sc as plsc

B = 16384
D = 128
NUID = 100000
LANES = 16
NC = 2
NS = 16
NW = NC * NS
BPW = B // NW
CHUNK = 128
NCH = BPW // CHUNK
EPV = D // LANES


def _sc_body(uidx_hbm, aidx_hbm, ut_hbm, bias_hbm, at_hbm,
             part_out, ub_out, ab_out,
             uidx_v, aidx_v, bidx_v, u0, u1, a0, a1, ubv, abv, accv,
             sem0, sem1, semb):
    wid = lax.axis_index("s") * NC + lax.axis_index("c")
    pltpu.sync_copy(uidx_hbm.at[wid], uidx_v)
    pltpu.sync_copy(aidx_hbm.at[wid], aidx_v)
    off = jnp.full((LANES,), NUID, jnp.int32)
    for ch in range(NCH):
        for k in range(CHUNK // LANES):
            sl = pl.ds(k * LANES, LANES)
            bidx_v[ch, sl] = aidx_v[ch, sl] + off

    ubufs = (u0, u1)
    abufs = (a0, a1)
    sems = (sem0, sem1)

    def fire(ch):
        return (pltpu.async_copy(ut_hbm.at[uidx_v.at[ch]], ubufs[ch % 2],
                                 sems[ch % 2]),
                pltpu.async_copy(at_hbm.at[aidx_v.at[ch]], abufs[ch % 2],
                                 sems[ch % 2]))

    pending = {0: fire(0)}

    bias_copies = []
    for ch in range(NCH):
        bias_copies.append(
            pltpu.async_copy(bias_hbm.at[uidx_v.at[ch]], ubv.at[ch], semb))
        bias_copies.append(
            pltpu.async_copy(bias_hbm.at[bidx_v.at[ch]], abv.at[ch], semb))

    accs = tuple(jnp.zeros((LANES,), jnp.float32) for _ in range(EPV))
    for ch in range(NCH):
        if ch + 1 < NCH:
            pending[ch + 1] = fire(ch + 1)
        cu, ca = pending.pop(ch)
        cu.wait()
        ca.wait()
        ubuf = ubufs[ch % 2]
        abuf = abufs[ch % 2]

        def row_body(r, accs, ubuf=ubuf, abuf=abuf):
            return tuple(
                accs[e] + (ubuf[r, pl.ds(e * LANES, LANES)] *
                           abuf[r, pl.ds(e * LANES, LANES)])
                for e in range(EPV))

        accs = lax.fori_loop(0, CHUNK, row_body, accs)

    acc = accs[0]
    for e in range(1, EPV):
        acc = acc + accs[e]
    accv[...] = acc
    pltpu.sync_copy(accv, part_out.at[wid])
    for c in bias_copies:
        c.wait()
    pltpu.sync_copy(ubv, ub_out.at[wid])
    pltpu.sync_copy(abv, ab_out.at[wid])


_sc_call = functools.partial(
    pl.kernel,
    mesh=plsc.VectorSubcoreMesh(core_axis_name="c", subcore_axis_name="s"),
    out_type=[
        jax.ShapeDtypeStruct((NW, LANES), jnp.float32),
        jax.ShapeDtypeStruct((NW, NCH, CHUNK), jnp.float32),
        jax.ShapeDtypeStruct((NW, NCH, CHUNK), jnp.float32),
    ],
    scratch_types=[
        pltpu.VMEM((NCH, CHUNK), jnp.int32),
        pltpu.VMEM((NCH, CHUNK), jnp.int32),
        pltpu.VMEM((NCH, CHUNK), jnp.int32),
        pltpu.VMEM((CHUNK, D), jnp.float32),
        pltpu.VMEM((CHUNK, D), jnp.float32),
        pltpu.VMEM((CHUNK, D), jnp.float32),
        pltpu.VMEM((CHUNK, D), jnp.float32),
        pltpu.VMEM((NCH, CHUNK), jnp.float32),
        pltpu.VMEM((NCH, CHUNK), jnp.float32),
        pltpu.VMEM((LANES,), jnp.float32),
        pltpu.SemaphoreType.DMA,
        pltpu.SemaphoreType.DMA,
        pltpu.SemaphoreType.DMA,
    ],
)(_sc_body)


def _combine(part_ref, ub_ref, ab_ref, o_ref):
    s = jnp.sum(part_ref[...])
    o_ref[...] = jax.nn.sigmoid(ub_ref[...] + ab_ref[...] + s)


def kernel(inputs, uid_table, uid_bias_table, aid_table, aid_bias_table):
    idx = inputs.astype(jnp.int32)
    uidx = idx[:, 0].reshape(NW, NCH, CHUNK)
    aidx = idx[:, 1].reshape(NW, NCH, CHUNK)
    bias = jnp.concatenate([uid_bias_table, aid_bias_table], axis=0)
    bias = bias.reshape(-1)

    part, ubg, abg = _sc_call(uidx, aidx, uid_table, bias, aid_table)

    out = pl.pallas_call(
        _combine,
        out_shape=jax.ShapeDtypeStruct((B // D, D), jnp.float32),
    )(part, ubg.reshape(B // D, D), abg.reshape(B // D, D))
    return out.reshape(B, 1)


# speedup vs baseline: 1.0950x; 1.0950x over previous
import functools

import jax
import jax.numpy as jnp
from jax import lax
from jax.experimental import pallas as pl
from jax.experimental.pallas import tpu as pltpu
from jax.experimental.pallas import tpu_sc as plsc

B = 16384
D = 128
LANES = 16
NC = 2
NS = 16
NW = NC * NS
BPW = B // NW
CHUNK = 128
NCH = BPW // CHUNK
EPV = D // LANES
NVEC = CHUNK // LANES


def _dot_body(uidx_hbm, aidx_hbm, ut_hbm, at_hbm,
              part_out,
              uidx_v, aidx_v, u0, u1, a0, a1, accv,
              sem0, sem1):
    wid = lax.axis_index("s") * NC + lax.axis_index("c")
    pltpu.sync_copy(uidx_hbm.at[wid], uidx_v)
    pltpu.sync_copy(aidx_hbm.at[wid], aidx_v)

    ubufs = (u0, u1)
    abufs = (a0, a1)
    sems = (sem0, sem1)

    def fire(ch):
        return (pltpu.async_copy(ut_hbm.at[uidx_v.at[ch]], ubufs[ch % 2],
                                 sems[ch % 2]),
                pltpu.async_copy(at_hbm.at[aidx_v.at[ch]], abufs[ch % 2],
                                 sems[ch % 2]))

    pending = {0: fire(0)}
    accs = tuple(jnp.zeros((LANES,), jnp.float32) for _ in range(EPV))
    for ch in range(NCH):
        if ch + 1 < NCH:
            pending[ch + 1] = fire(ch + 1)
        cu, ca = pending.pop(ch)
        cu.wait()
        ca.wait()
        ubuf = ubufs[ch % 2]
        abuf = abufs[ch % 2]

        def row_body(r, accs, ubuf=ubuf, abuf=abuf):
            return tuple(
                accs[e] + (ubuf[r, pl.ds(e * LANES, LANES)] *
                           abuf[r, pl.ds(e * LANES, LANES)])
                for e in range(EPV))

        accs = lax.fori_loop(0, CHUNK, row_body, accs)

    acc = accs[0]
    for e in range(1, EPV):
        acc = acc + accs[e]
    accv[...] = acc
    pltpu.sync_copy(accv, part_out.at[wid])


_dot_call = functools.partial(
    pl.kernel,
    mesh=plsc.VectorSubcoreMesh(core_axis_name="c", subcore_axis_name="s"),
    out_type=[
        jax.ShapeDtypeStruct((NW, LANES), jnp.float32),
    ],
    scratch_types=[
        pltpu.VMEM((NCH, CHUNK), jnp.int32),
        pltpu.VMEM((NCH, CHUNK), jnp.int32),
        pltpu.VMEM((CHUNK, D), jnp.float32),
        pltpu.VMEM((CHUNK, D), jnp.float32),
        pltpu.VMEM((CHUNK, D), jnp.float32),
        pltpu.VMEM((CHUNK, D), jnp.float32),
        pltpu.VMEM((LANES,), jnp.float32),
        pltpu.SemaphoreType.DMA,
        pltpu.SemaphoreType.DMA,
    ],
)(_dot_body)


def _bias_body(uidx_hbm, aidx_hbm, ub_hbm, ab_hbm, part_hbm,
               out_hbm,
               uidx_v, aidx_v, ubv, abv, partv, outv, semb):
    wid = lax.axis_index("s") * NC + lax.axis_index("c")
    pltpu.sync_copy(uidx_hbm.at[wid], uidx_v)
    pltpu.sync_copy(aidx_hbm.at[wid], aidx_v)

    bias_copies = []
    for ch in range(NCH):
        bias_copies.append(
            pltpu.async_copy(ub_hbm.at[uidx_v.at[ch]], ubv.at[ch], semb))
        bias_copies.append(
            pltpu.async_copy(ab_hbm.at[aidx_v.at[ch]], abv.at[ch], semb))

    pltpu.sync_copy(part_hbm, partv)
    s = partv[0, :]
    for w in range(1, NW):
        s = s + partv[w, :]
    for sh in (8, 4, 2, 1):
        perm = (lax.iota(jnp.int32, LANES) + sh) & (LANES - 1)
        s = s + s.at[perm].get(mode="promise_in_bounds")

    for c in bias_copies:
        c.wait()
    for ch in range(NCH):
        for k in range(NVEC):
            sl = pl.ds(k * LANES, LANES)
            x = ubv[ch, sl] + abv[ch, sl] + s
            outv[ch, sl] = 1.0 / (1.0 + jnp.exp(-x))
    pltpu.sync_copy(outv, out_hbm.at[wid])


_bias_call = functools.partial(
    pl.kernel,
    mesh=plsc.VectorSubcoreMesh(core_axis_name="c", subcore_axis_name="s"),
    out_type=[
        jax.ShapeDtypeStruct((NW, NCH, CHUNK), jnp.float32),
    ],
    scratch_types=[
        pltpu.VMEM((NCH, CHUNK), jnp.int32),
        pltpu.VMEM((NCH, CHUNK), jnp.int32),
        pltpu.VMEM((NCH, CHUNK), jnp.float32),
        pltpu.VMEM((NCH, CHUNK), jnp.float32),
        pltpu.VMEM((NW, LANES), jnp.float32),
        pltpu.VMEM((NCH, CHUNK), jnp.float32),
        pltpu.SemaphoreType.DMA,
    ],
)(_bias_body)


def kernel(inputs, uid_table, uid_bias_table, aid_table, aid_bias_table):
    idx = inputs.astype(jnp.int32)
    uidx = idx[:, 0].reshape(NW, NCH, CHUNK)
    aidx = idx[:, 1].reshape(NW, NCH, CHUNK)
    ub1 = uid_bias_table.reshape(-1)
    ab1 = aid_bias_table.reshape(-1)

    (part,) = _dot_call(uidx, aidx, uid_table, aid_table)
    (out,) = _bias_call(uidx, aidx, ub1, ab1, part)
    return out.reshape(B, 1)


# speedup vs baseline: 1.1150x; 1.0183x over previous
import functools

import jax
import jax.numpy as jnp
from jax import lax
from jax.experimental import pallas as pl
from jax.experimental.pallas import tpu as pltpu
from jax.experimental.pallas import tpu_sc as plsc

B = 16384
D = 128
LANES = 16
NC = 2
NS = 16
NW = NC * NS
BPW = B // NW
CHUNK = 128
NCH = BPW // CHUNK
EPV = D // LANES
NVEC = CHUNK // LANES
NBUF = 3


def _dot_body(in3_hbm, ut_hbm, at_hbm,
              part_out,
              idx_v, u0, u1, u2, a0, a1, a2, accv,
              sem0, sem1, sem2):
    wid = lax.axis_index("s") * NC + lax.axis_index("c")
    pltpu.sync_copy(in3_hbm.at[pl.ds(NCH * wid, NCH)], idx_v)

    ubufs = (u0, u1, u2)
    abufs = (a0, a1, a2)
    sems = (sem0, sem1, sem2)

    def fire(ch):
        b = ch % NBUF
        return (pltpu.async_copy(ut_hbm.at[idx_v.at[ch, 0]], ubufs[b],
                                 sems[b]),
                pltpu.async_copy(at_hbm.at[idx_v.at[ch, 1]], abufs[b],
                                 sems[b]))

    pending = {ch: fire(ch) for ch in range(min(NBUF, NCH))}
    accs = tuple(jnp.zeros((LANES,), jnp.float32) for _ in range(EPV))
    for ch in range(NCH):
        cu, ca = pending.pop(ch)
        cu.wait()
        ca.wait()
        ubuf = ubufs[ch % NBUF]
        abuf = abufs[ch % NBUF]

        def row_body(r, accs, ubuf=ubuf, abuf=abuf):
            return tuple(
                accs[e] + (ubuf[r, pl.ds(e * LANES, LANES)] *
                           abuf[r, pl.ds(e * LANES, LANES)])
                for e in range(EPV))

        accs = lax.fori_loop(0, CHUNK, row_body, accs)
        if ch + NBUF < NCH:
            pending[ch + NBUF] = fire(ch + NBUF)

    acc = accs[0]
    for e in range(1, EPV):
        acc = acc + accs[e]
    accv[...] = acc
    pltpu.sync_copy(accv, part_out.at[wid])


_dot_call = functools.partial(
    pl.kernel,
    mesh=plsc.VectorSubcoreMesh(core_axis_name="c", subcore_axis_name="s"),
    out_type=[
        jax.ShapeDtypeStruct((NW, LANES), jnp.float32),
    ],
    scratch_types=[
        pltpu.VMEM((NCH, 2, CHUNK), jnp.int32),
        pltpu.VMEM((CHUNK, D), jnp.float32),
        pltpu.VMEM((CHUNK, D), jnp.float32),
        pltpu.VMEM((CHUNK, D), jnp.float32),
        pltpu.VMEM((CHUNK, D), jnp.float32),
        pltpu.VMEM((CHUNK, D), jnp.float32),
        pltpu.VMEM((CHUNK, D), jnp.float32),
        pltpu.VMEM((LANES,), jnp.float32),
        pltpu.SemaphoreType.DMA,
        pltpu.SemaphoreType.DMA,
        pltpu.SemaphoreType.DMA,
    ],
)(_dot_body)


def _bias_body(in3_hbm, ub_hbm, ab_hbm, part_hbm,
               out_hbm,
               idx_v, ubv, abv, partv, outv, semb):
    wid = lax.axis_index("s") * NC + lax.axis_index("c")
    pltpu.sync_copy(in3_hbm.at[pl.ds(NCH * wid, NCH)], idx_v)

    bias_copies = []
    for ch in range(NCH):
        bias_copies.append(
            pltpu.async_copy(ub_hbm.at[idx_v.at[ch, 0]], ubv.at[ch], semb))
        bias_copies.append(
            pltpu.async_copy(ab_hbm.at[idx_v.at[ch, 1]], abv.at[ch], semb))

    pltpu.sync_copy(part_hbm, partv)
    s = partv[0, :]
    for w in range(1, NW):
        s = s + partv[w, :]
    for sh in (8, 4, 2, 1):
        perm = (lax.iota(jnp.int32, LANES) + sh) & (LANES - 1)
        s = s + s.at[perm].get(mode="promise_in_bounds")

    for c in bias_copies:
        c.wait()
    for ch in range(NCH):
        for k in range(NVEC):
            sl = pl.ds(k * LANES, LANES)
            x = ubv[ch, sl] + abv[ch, sl] + s
            outv[ch, sl] = 1.0 / (1.0 + jnp.exp(-x))
    pltpu.sync_copy(outv, out_hbm.at[wid])


_bias_call = functools.partial(
    pl.kernel,
    mesh=plsc.VectorSubcoreMesh(core_axis_name="c", subcore_axis_name="s"),
    out_type=[
        jax.ShapeDtypeStruct((NW, NCH, CHUNK), jnp.float32),
    ],
    scratch_types=[
        pltpu.VMEM((NCH, 2, CHUNK), jnp.int32),
        pltpu.VMEM((NCH, CHUNK), jnp.float32),
        pltpu.VMEM((NCH, CHUNK), jnp.float32),
        pltpu.VMEM((NW, LANES), jnp.float32),
        pltpu.VMEM((NCH, CHUNK), jnp.float32),
        pltpu.SemaphoreType.DMA,
    ],
)(_bias_body)


def kernel(inputs, uid_table, uid_bias_table, aid_table, aid_bias_table):
    idx = inputs.astype(jnp.int32)
    in3 = idx.T.reshape(2, B // CHUNK, CHUNK).transpose(1, 0, 2)
    ub1 = uid_bias_table.reshape(-1)
    ab1 = aid_bias_table.reshape(-1)

    (part,) = _dot_call(in3, uid_table, aid_table)
    (out,) = _bias_call(in3, ub1, ab1, part)
    return out.reshape(B, 1)
